# Initial kernel scaffold; baseline (speedup 1.0000x reference)
import functools

import jax
import jax.numpy as jnp
from jax import lax
from jax.experimental import pallas as pl
from jax.experimental.pallas import tpu as pltpu
from jax.experimental.pallas import tpu_sc as plsc

N_FIELDS = 26
VOCAB = 100000
EMB_DIM = 16
NUM_DIM = 13
BATCH = 16384

NUM_WORKERS = 32
N_CHUNKS = 16
CHUNK = BATCH // N_CHUNKS
TASKS_PER_TILE = (N_FIELDS * N_CHUNKS) // NUM_WORKERS


def _gather_body(cats_hbm, tables_hbm, out_hbm, idx_v, rows_v, sem):
    wid = lax.axis_index("s") * 2 + lax.axis_index("c")

    def body(t, carry):
        task = wid * TASKS_PER_TILE + t
        f = task // N_CHUNKS
        c = task % N_CHUNKS
        base = c * CHUNK
        pltpu.sync_copy(cats_hbm.at[f, pl.ds(base, CHUNK)], idx_v)
        pltpu.async_copy(tables_hbm.at[idx_v], rows_v, sem).wait()
        pltpu.sync_copy(rows_v, out_hbm.at[pl.ds(base, CHUNK), f])
        return carry

    lax.fori_loop(0, TASKS_PER_TILE, body, 0)


_gather = functools.partial(
    pl.kernel,
    out_type=jax.ShapeDtypeStruct((BATCH, N_FIELDS, EMB_DIM), jnp.float32),
    mesh=plsc.VectorSubcoreMesh(core_axis_name="c", subcore_axis_name="s"),
    scratch_types=[
        pltpu.VMEM((CHUNK,), jnp.int32),
        pltpu.VMEM((CHUNK, EMB_DIM), jnp.float32),
        pltpu.SemaphoreType.DMA,
    ],
)(_gather_body)


def _mlp_body(cat_ref, nums_ref, w1a_ref, w1n_ref, b1_ref, w2_ref, b2_ref,
              out_ref):
    h = cat_ref[...] @ w1a_ref[...] + nums_ref[...] @ w1n_ref[...] + b1_ref[...]
    h = jnp.maximum(h, 0.0)
    out_ref[...] = h @ w2_ref[...] + b2_ref[...]


def _mlp(cat_feat, nums, w1a, w1n, b1, w2, b2):
    bm = 2048
    grid = BATCH // bm
    return pl.pallas_call(
        _mlp_body,
        grid=(grid,),
        in_specs=[
            pl.BlockSpec((bm, N_FIELDS * EMB_DIM), lambda i: (i, 0)),
            pl.BlockSpec((bm, NUM_DIM), lambda i: (i, 0)),
            pl.BlockSpec((N_FIELDS * EMB_DIM, 64), lambda i: (0, 0)),
            pl.BlockSpec((NUM_DIM, 64), lambda i: (0, 0)),
            pl.BlockSpec((1, 64), lambda i: (0, 0)),
            pl.BlockSpec((64, 1), lambda i: (0, 0)),
            pl.BlockSpec((1, 1), lambda i: (0, 0)),
        ],
        out_specs=pl.BlockSpec((bm, 1), lambda i: (i, 0)),
        out_shape=jax.ShapeDtypeStruct((BATCH, 1), jnp.float32),
    )(cat_feat, nums, w1a, w1n, b1, w2, b2)


def kernel(cats, nums, emb_tables, W1, b1, W2, b2):
    offs = (jnp.arange(N_FIELDS, dtype=jnp.int32) * (VOCAB + 1))[:, None]
    cats32 = cats.astype(jnp.int32) + offs
    flat_tables = emb_tables.reshape(N_FIELDS * (VOCAB + 1), EMB_DIM)
    embs = _gather(cats32, flat_tables)
    cat_feat = embs.reshape(BATCH, N_FIELDS * EMB_DIM)
    w1a = W1[: N_FIELDS * EMB_DIM]
    w1n = W1[N_FIELDS * EMB_DIM :]
    return _mlp(cat_feat, nums, w1a, w1n, b1[None, :], W2, b2[None, :])


# speedup vs baseline: 1.2499x; 1.2499x over previous
import functools

import jax
import jax.numpy as jnp
from jax import lax
from jax.experimental import pallas as pl
from jax.experimental.pallas import tpu as pltpu
from jax.experimental.pallas import tpu_sc as plsc

N_FIELDS = 26
VOCAB = 100000
EMB_DIM = 16
NUM_DIM = 13
BATCH = 16384

NUM_WORKERS = 32
N_CHUNKS = 16
CHUNK = BATCH // N_CHUNKS
TASKS_PER_TILE = (N_FIELDS * N_CHUNKS) // NUM_WORKERS


def _gather_body(cats_hbm, tables_hbm, out_hbm, idx_v, rows_v, sem):
    wid = lax.axis_index("s") * 2 + lax.axis_index("c")

    def body(t, carry):
        task = wid * TASKS_PER_TILE + t
        f = task // N_CHUNKS
        c = task % N_CHUNKS
        base = c * CHUNK
        pltpu.sync_copy(cats_hbm.at[f, pl.ds(base, CHUNK)], idx_v)
        pltpu.async_copy(tables_hbm.at[idx_v], rows_v, sem).wait()
        pltpu.sync_copy(rows_v, out_hbm.at[pl.ds(base, CHUNK), f])
        return carry

    lax.fori_loop(0, TASKS_PER_TILE, body, 0)


_gather = functools.partial(
    pl.kernel,
    out_type=jax.ShapeDtypeStruct((BATCH, N_FIELDS, EMB_DIM), jnp.float32),
    mesh=plsc.VectorSubcoreMesh(core_axis_name="c", subcore_axis_name="s"),
    scratch_types=[
        pltpu.VMEM((CHUNK,), jnp.int32),
        pltpu.VMEM((CHUNK, EMB_DIM), jnp.float32),
        pltpu.SemaphoreType.DMA,
    ],
    compiler_params=pltpu.CompilerParams(use_tc_tiling_on_sc=False),
)(_gather_body)


def _mlp_body(cat_ref, nums_ref, w1a_ref, w1n_ref, b1_ref, w2_ref, b2_ref,
              out_ref):
    h = cat_ref[...] @ w1a_ref[...] + nums_ref[...] @ w1n_ref[...] + b1_ref[...]
    h = jnp.maximum(h, 0.0)
    out_ref[...] = h @ w2_ref[...] + b2_ref[...]


def _mlp(cat_feat, nums, w1a, w1n, b1, w2, b2):
    bm = 2048
    grid = BATCH // bm
    return pl.pallas_call(
        _mlp_body,
        grid=(grid,),
        in_specs=[
            pl.BlockSpec((bm, N_FIELDS * EMB_DIM), lambda i: (i, 0)),
            pl.BlockSpec((bm, NUM_DIM), lambda i: (i, 0)),
            pl.BlockSpec((N_FIELDS * EMB_DIM, 64), lambda i: (0, 0)),
            pl.BlockSpec((NUM_DIM, 64), lambda i: (0, 0)),
            pl.BlockSpec((1, 64), lambda i: (0, 0)),
            pl.BlockSpec((64, 1), lambda i: (0, 0)),
            pl.BlockSpec((1, 1), lambda i: (0, 0)),
        ],
        out_specs=pl.BlockSpec((bm, 1), lambda i: (i, 0)),
        out_shape=jax.ShapeDtypeStruct((BATCH, 1), jnp.float32),
    )(cat_feat, nums, w1a, w1n, b1, w2, b2)


def kernel(cats, nums, emb_tables, W1, b1, W2, b2):
    offs = (jnp.arange(N_FIELDS, dtype=jnp.int32) * (VOCAB + 1))[:, None]
    cats32 = cats.astype(jnp.int32) + offs
    flat_tables = emb_tables.reshape(N_FIELDS * (VOCAB + 1), EMB_DIM)
    embs = _gather(cats32, flat_tables)
    cat_feat = embs.reshape(BATCH, N_FIELDS * EMB_DIM)
    w1a = W1[: N_FIELDS * EMB_DIM]
    w1n = W1[N_FIELDS * EMB_DIM :]
    return _mlp(cat_feat, nums, w1a, w1n, b1[None, :], W2, b2[None, :])


# speedup vs baseline: 4.0456x; 3.2366x over previous
import functools

import jax
import jax.numpy as jnp
from jax import lax
from jax.experimental import pallas as pl
from jax.experimental.pallas import tpu as pltpu
from jax.experimental.pallas import tpu_sc as plsc

N_FIELDS = 26
VOCAB = 100000
EMB_DIM = 16
NUM_DIM = 13
BATCH = 16384

VPAD = 100352
VBLK = 1024
N_VBLK = VPAD // VBLK

NUM_WORKERS = 32
N_CHUNKS = 16
CHUNK = BATCH // N_CHUNKS
TASKS_PER_TILE = (N_FIELDS * N_CHUNKS) // NUM_WORKERS


def _reformat_body(tab_ref, out_ref):
    x = tab_ref[0]
    eye = jnp.eye(EMB_DIM, dtype=jnp.float32)
    y = jax.lax.dot_general(x, eye, (((0,), (0,)), ((), ())),
                            preferred_element_type=jnp.float32)
    y8 = y.reshape(VBLK // 8, 8, EMB_DIM)
    z = jnp.concatenate([y8[:, h, :] for h in range(8)], axis=1)
    out_ref[...] = z.reshape(VBLK * EMB_DIM // 1024, 8, 128)


def _reformat(tab_t):
    m_blk = VBLK * EMB_DIM // 1024
    return pl.pallas_call(
        _reformat_body,
        grid=(N_FIELDS, N_VBLK),
        in_specs=[pl.BlockSpec((1, EMB_DIM, VBLK), lambda f, j: (f, 0, j))],
        out_specs=pl.BlockSpec((m_blk, 8, 128),
                               lambda f, j: (f * N_VBLK + j, 0, 0)),
        out_shape=jax.ShapeDtypeStruct(
            (N_FIELDS * N_VBLK * m_blk, 8, 128), jnp.float32),
    )(tab_t)


def _gather_body(cats_hbm, tables_hbm, out_hbm, idx_v, rows_v, sem):
    wid = lax.axis_index("s") * 2 + lax.axis_index("c")

    def body(t, carry):
        task = wid * TASKS_PER_TILE + t
        f = task // N_CHUNKS
        c = task % N_CHUNKS
        base = c * CHUNK
        pltpu.sync_copy(cats_hbm.at[f, pl.ds(base, CHUNK)], idx_v)
        pltpu.async_copy(tables_hbm.at[idx_v], rows_v, sem).wait()
        pltpu.sync_copy(rows_v, out_hbm.at[pl.ds(base, CHUNK), f])
        return carry

    lax.fori_loop(0, TASKS_PER_TILE, body, 0)


_gather = functools.partial(
    pl.kernel,
    out_type=jax.ShapeDtypeStruct((BATCH, N_FIELDS, EMB_DIM), jnp.float32),
    mesh=plsc.VectorSubcoreMesh(core_axis_name="c", subcore_axis_name="s"),
    scratch_types=[
        pltpu.VMEM((CHUNK,), jnp.int32),
        pltpu.VMEM((CHUNK, EMB_DIM), jnp.float32),
        pltpu.SemaphoreType.DMA,
    ],
    compiler_params=pltpu.CompilerParams(use_tc_tiling_on_sc=False),
)(_gather_body)


def _mlp_body(cat_ref, nums_ref, w1a_ref, w1n_ref, b1_ref, w2_ref, b2_ref,
              out_ref):
    h = cat_ref[...] @ w1a_ref[...] + nums_ref[...] @ w1n_ref[...] + b1_ref[...]
    h = jnp.maximum(h, 0.0)
    out_ref[...] = h @ w2_ref[...] + b2_ref[...]


def _mlp(cat_feat, nums, w1a, w1n, b1, w2, b2):
    bm = 2048
    grid = BATCH // bm
    return pl.pallas_call(
        _mlp_body,
        grid=(grid,),
        in_specs=[
            pl.BlockSpec((bm, N_FIELDS * EMB_DIM), lambda i: (i, 0)),
            pl.BlockSpec((bm, NUM_DIM), lambda i: (i, 0)),
            pl.BlockSpec((N_FIELDS * EMB_DIM, 64), lambda i: (0, 0)),
            pl.BlockSpec((NUM_DIM, 64), lambda i: (0, 0)),
            pl.BlockSpec((1, 64), lambda i: (0, 0)),
            pl.BlockSpec((64, 1), lambda i: (0, 0)),
            pl.BlockSpec((1, 1), lambda i: (0, 0)),
        ],
        out_specs=pl.BlockSpec((bm, 1), lambda i: (i, 0)),
        out_shape=jax.ShapeDtypeStruct((BATCH, 1), jnp.float32),
    )(cat_feat, nums, w1a, w1n, b1, w2, b2)


def kernel(cats, nums, emb_tables, W1, b1, W2, b2):
    offs = (jnp.arange(N_FIELDS, dtype=jnp.int32) * VPAD)[:, None]
    cats32 = cats.astype(jnp.int32) + offs
    tab_t = jnp.transpose(emb_tables, (0, 2, 1))
    flat_tables = _reformat(tab_t).reshape(N_FIELDS * VPAD, EMB_DIM)
    embs = _gather(cats32, flat_tables)
    cat_feat = embs.reshape(BATCH, N_FIELDS * EMB_DIM)
    w1a = W1[: N_FIELDS * EMB_DIM]
    w1n = W1[N_FIELDS * EMB_DIM :]
    return _mlp(cat_feat, nums, w1a, w1n, b1[None, :], W2, b2[None, :])


# speedup vs baseline: 5.8136x; 1.4370x over previous
import functools

import jax
import jax.numpy as jnp
from jax import lax
from jax.experimental import pallas as pl
from jax.experimental.pallas import tpu as pltpu
from jax.experimental.pallas import tpu_sc as plsc

N_FIELDS = 26
VOCAB = 100000
EMB_DIM = 16
NUM_DIM = 13
BATCH = 16384

VPAD = 100096
TAIL0 = 99968

NUM_WORKERS = 32
N_CHUNKS = 16
CHUNK = BATCH // N_CHUNKS
TASKS_PER_TILE = (N_FIELDS * N_CHUNKS) // NUM_WORKERS

BULK_CHUNKS = 97
TOT_BULK = N_FIELDS * BULK_CHUNKS


def _tail_body(tab_ref, out_ref):
    x = tab_ref[0]
    eye = jnp.eye(EMB_DIM, dtype=jnp.float32)
    y = jax.lax.dot_general(x, eye, (((0,), (0,)), ((), ())),
                            preferred_element_type=jnp.float32)
    y8 = y.reshape(16, 8, EMB_DIM)
    out_ref[...] = jnp.concatenate([y8[:, h, :] for h in range(8)], axis=1)


def _tail(tab_t):
    return pl.pallas_call(
        _tail_body,
        grid=(N_FIELDS,),
        in_specs=[pl.BlockSpec((1, EMB_DIM, 128),
                               lambda f: (f, 0, TAIL0 // 128))],
        out_specs=pl.BlockSpec((16, 128), lambda f: (f, 0)),
        out_shape=jax.ShapeDtypeStruct((N_FIELDS * 16, 128), jnp.float32),
    )(tab_t)


def _reformat_body(tab_hbm, tail_hbm, cats_hbm, ltab_hbm, lcats_hbm,
                   inb, outb, rowb, tailb, sem):
    wid = lax.axis_index("s") * 2 + lax.axis_index("c")
    lanes = lax.iota(jnp.int32, 16)

    def put_row(v, x):
        plsc.store_scatter(
            outb, [jnp.full((16,), v // 64, jnp.int32),
                   jnp.full((16,), (v % 64) // 8, jnp.int32),
                   (v % 8) * 16 + lanes], x)

    def do_chunk(f, col0, width):
        pltpu.sync_copy(tab_hbm.at[f, :, pl.ds(col0, width)],
                        inb.at[:, pl.ds(0, width)])

        def row(v, _):
            put_row(v, plsc.load_gather(
                inb, [lanes, jnp.full((16,), v, jnp.int32)]))
            return 0

        lax.fori_loop(0, width, row, 0, unroll=8)
        nblk = width * EMB_DIM // 1024
        pltpu.sync_copy(outb.at[pl.ds(0, nblk)],
                        ltab_hbm.at[pl.ds((f * VPAD + col0) // 64, nblk)])

    def bulk_task(t, _):
        q = t * NUM_WORKERS + wid

        @pl.when(q < TOT_BULK)
        def _():
            f = q // BULK_CHUNKS
            j = q % BULK_CHUNKS
            do_chunk(f, j * 1024, 1024)

        return 0

    lax.fori_loop(0, (TOT_BULK + NUM_WORKERS - 1) // NUM_WORKERS, bulk_task, 0)

    @pl.when(wid < N_FIELDS)
    def _():
        f = wid
        do_chunk(f, BULK_CHUNKS * 1024, TAIL0 - BULK_CHUNKS * 1024)
        pltpu.sync_copy(tail_hbm.at[pl.ds(f * 16, 16)], tailb)

        def trow(v, _):
            put_row(v, plsc.load_gather(
                tailb, [jnp.full((16,), v // 8, jnp.int32),
                        (v % 8) * 16 + lanes]))
            return 0

        lax.fori_loop(0, 128, trow, 0, unroll=8)
        pltpu.sync_copy(outb.at[pl.ds(0, 2)],
                        ltab_hbm.at[pl.ds((f * VPAD + TAIL0) // 64, 2)])
        pltpu.sync_copy(cats_hbm.at[f], rowb)

        def add_off(k, _):
            rowb[pl.ds(k * 16, 16)] = rowb[pl.ds(k * 16, 16)] + f * VPAD
            return 0

        lax.fori_loop(0, BATCH // 16, add_off, 0, unroll=8)
        pltpu.sync_copy(rowb, lcats_hbm.at[pl.ds(f * BATCH, BATCH)])


_reformat = functools.partial(
    pl.kernel,
    out_type=(
        jax.ShapeDtypeStruct((N_FIELDS * VPAD * EMB_DIM // 1024, 8, 128),
                             jnp.float32),
        jax.ShapeDtypeStruct((N_FIELDS * BATCH,), jnp.int32),
    ),
    mesh=plsc.VectorSubcoreMesh(core_axis_name="c", subcore_axis_name="s"),
    scratch_types=[
        pltpu.VMEM((EMB_DIM, 1024), jnp.float32),
        pltpu.VMEM((16, 8, 128), jnp.float32),
        pltpu.VMEM((BATCH,), jnp.int32),
        pltpu.VMEM((16, 128), jnp.float32),
        pltpu.SemaphoreType.DMA,
    ],
    compiler_params=pltpu.CompilerParams(needs_layout_passes=False),
)(_reformat_body)


def _gather_body(lcats_hbm, ltab_hbm, out_hbm, idx_v, rows_v, sem):
    wid = lax.axis_index("s") * 2 + lax.axis_index("c")

    def body(t, carry):
        task = wid * TASKS_PER_TILE + t
        f = task // N_CHUNKS
        c = task % N_CHUNKS
        base = c * CHUNK
        pltpu.sync_copy(lcats_hbm.at[pl.ds(f * BATCH + base, CHUNK)], idx_v)
        pltpu.async_copy(ltab_hbm.at[idx_v], rows_v, sem).wait()
        pltpu.sync_copy(rows_v, out_hbm.at[pl.ds(base, CHUNK), f])
        return carry

    lax.fori_loop(0, TASKS_PER_TILE, body, 0)


_gather = functools.partial(
    pl.kernel,
    out_type=jax.ShapeDtypeStruct((BATCH, N_FIELDS, EMB_DIM), jnp.float32),
    mesh=plsc.VectorSubcoreMesh(core_axis_name="c", subcore_axis_name="s"),
    scratch_types=[
        pltpu.VMEM((CHUNK,), jnp.int32),
        pltpu.VMEM((CHUNK, EMB_DIM), jnp.float32),
        pltpu.SemaphoreType.DMA,
    ],
    compiler_params=pltpu.CompilerParams(use_tc_tiling_on_sc=False),
)(_gather_body)


def _mlp_body(cat_ref, nums_ref, w1a_ref, w1n_ref, b1_ref, w2_ref, b2_ref,
              out_ref):
    h = cat_ref[...] @ w1a_ref[...] + nums_ref[...] @ w1n_ref[...] + b1_ref[...]
    h = jnp.maximum(h, 0.0)
    out_ref[...] = h @ w2_ref[...] + b2_ref[...]


def _mlp(cat_feat, nums, w1a, w1n, b1, w2, b2):
    bm = 2048
    grid = BATCH // bm
    return pl.pallas_call(
        _mlp_body,
        grid=(grid,),
        in_specs=[
            pl.BlockSpec((bm, N_FIELDS * EMB_DIM), lambda i: (i, 0)),
            pl.BlockSpec((bm, NUM_DIM), lambda i: (i, 0)),
            pl.BlockSpec((N_FIELDS * EMB_DIM, 64), lambda i: (0, 0)),
            pl.BlockSpec((NUM_DIM, 64), lambda i: (0, 0)),
            pl.BlockSpec((1, 64), lambda i: (0, 0)),
            pl.BlockSpec((64, 1), lambda i: (0, 0)),
            pl.BlockSpec((1, 1), lambda i: (0, 0)),
        ],
        out_specs=pl.BlockSpec((bm, 1), lambda i: (i, 0)),
        out_shape=jax.ShapeDtypeStruct((BATCH, 1), jnp.float32),
    )(cat_feat, nums, w1a, w1n, b1, w2, b2)


def kernel(cats, nums, emb_tables, W1, b1, W2, b2):
    tab_t = jnp.transpose(emb_tables, (0, 2, 1))
    tail = _tail(tab_t)
    ltab3, lcats = _reformat(tab_t, tail, cats.astype(jnp.int32))
    ltab = ltab3.reshape(N_FIELDS * VPAD, EMB_DIM)
    embs = _gather(lcats, ltab)
    cat_feat = embs.reshape(BATCH, N_FIELDS * EMB_DIM)
    w1a = W1[: N_FIELDS * EMB_DIM]
    w1n = W1[N_FIELDS * EMB_DIM :]
    return _mlp(cat_feat, nums, w1a, w1n, b1[None, :], W2, b2[None, :])


# speedup vs baseline: 8.1348x; 1.3993x over previous
import functools

import jax
import jax.numpy as jnp
from jax import lax
from jax.experimental import pallas as pl
from jax.experimental.pallas import tpu as pltpu
from jax.experimental.pallas import tpu_sc as plsc

N_FIELDS = 26
VOCAB = 100000
EMB_DIM = 16
NUM_DIM = 13
BATCH = 16384

VPAD = 100096
TAIL0 = 99968

NUM_WORKERS = 32
N_CHUNKS = 16
CHUNK = BATCH // N_CHUNKS
TASKS_PER_TILE = (N_FIELDS * N_CHUNKS) // NUM_WORKERS

BULK_CHUNKS = 97
TOT_BULK = N_FIELDS * BULK_CHUNKS
BULK_STEPS = (TOT_BULK + NUM_WORKERS - 1) // NUM_WORKERS


def _tail_body(tab_ref, out_ref):
    x = tab_ref[0]
    eye = jnp.eye(EMB_DIM, dtype=jnp.float32)
    y = jax.lax.dot_general(x, eye, (((0,), (0,)), ((), ())),
                            preferred_element_type=jnp.float32)
    y8 = y.reshape(16, 8, EMB_DIM)
    out_ref[...] = jnp.concatenate([y8[:, h, :] for h in range(8)], axis=1)


def _tail(tab_t):
    return pl.pallas_call(
        _tail_body,
        grid=(N_FIELDS,),
        in_specs=[pl.BlockSpec((1, EMB_DIM, 128),
                               lambda f: (f, 0, TAIL0 // 128))],
        out_specs=pl.BlockSpec((16, 128), lambda f: (f, 0)),
        out_shape=jax.ShapeDtypeStruct((N_FIELDS * 16, 128), jnp.float32),
    )(tab_t)


def _reformat_body(tab_hbm, tail_hbm, cats_hbm, ltab_hbm, lcats_hbm,
                   inb0, inb1, outb0, outb1, rowb, tailb,
                   sin0, sin1, sout0, sout1):
    wid = lax.axis_index("s") * 2 + lax.axis_index("c")
    lanes = lax.iota(jnp.int32, 16)

    def chunk_q(t):
        return jnp.minimum(t * NUM_WORKERS + wid, TOT_BULK - 1)

    def start_in(t, ib, sem):
        q = chunk_q(t)
        f = q // BULK_CHUNKS
        col0 = (q % BULK_CHUNKS) * 1024
        pltpu.async_copy(tab_hbm.at[f, :, pl.ds(col0, 1024)], ib, sem)

    def wait_in(sem):
        pltpu.make_async_copy(tab_hbm.at[0, :, pl.ds(0, 1024)], inb0,
                              sem).wait()

    def extract(ib, ob, width):
        def row(v, _):
            x = plsc.load_gather(ib, [lanes, jnp.full((16,), v, jnp.int32)])
            ob[pl.ds(v * EMB_DIM, EMB_DIM)] = x
            return 0

        lax.fori_loop(0, width, row, 0, unroll=8)

    def start_out(t, ob, sem):
        q = chunk_q(t)
        f = q // BULK_CHUNKS
        col0 = (q % BULK_CHUNKS) * 1024
        dst = (f * VPAD + col0) * EMB_DIM
        pltpu.async_copy(ob, ltab_hbm.at[pl.ds(dst, 1024 * EMB_DIM)], sem)

    def wait_out(sem):
        pltpu.make_async_copy(outb0, ltab_hbm.at[pl.ds(0, 1024 * EMB_DIM)],
                              sem).wait()

    start_in(0, inb0, sin0)

    def phase(t, ib, ob, sin, sout, ib_next, sin_next):
        @pl.when(t + 1 < BULK_STEPS)
        def _():
            start_in(t + 1, ib_next, sin_next)

        wait_in(sin)

        @pl.when(t >= 2)
        def _():
            wait_out(sout)

        extract(ib, ob, 1024)
        start_out(t, ob, sout)

    def body(t, _):
        @pl.when(t % 2 == 0)
        def _():
            phase(t, inb0, outb0, sin0, sout0, inb1, sin1)

        @pl.when(t % 2 == 1)
        def _():
            phase(t, inb1, outb1, sin1, sout1, inb0, sin0)

        return 0

    lax.fori_loop(0, BULK_STEPS, body, 0)
    wait_out(sout0)
    wait_out(sout1)

    @pl.when(wid < N_FIELDS)
    def _():
        f = wid
        w = TAIL0 - BULK_CHUNKS * 1024
        pltpu.async_copy(tab_hbm.at[f, :, pl.ds(BULK_CHUNKS * 1024, w)],
                         inb0.at[:, pl.ds(0, w)], sin0)
        pltpu.make_async_copy(tab_hbm.at[0, :, pl.ds(0, w)],
                              inb0.at[:, pl.ds(0, w)], sin0).wait()
        extract(inb0, outb0, w)
        pltpu.async_copy(outb0.at[pl.ds(0, w * EMB_DIM)],
                         ltab_hbm.at[pl.ds((f * VPAD + BULK_CHUNKS * 1024)
                                           * EMB_DIM, w * EMB_DIM)], sout0)
        pltpu.sync_copy(tail_hbm.at[pl.ds(f * 16, 16)], tailb)

        def trow(v, _):
            x = plsc.load_gather(
                tailb, [jnp.full((16,), v // 8, jnp.int32),
                        (v % 8) * EMB_DIM + lanes])
            outb1[pl.ds(v * EMB_DIM, EMB_DIM)] = x
            return 0

        lax.fori_loop(0, 128, trow, 0, unroll=8)
        pltpu.async_copy(outb1.at[pl.ds(0, 128 * EMB_DIM)],
                         ltab_hbm.at[pl.ds((f * VPAD + TAIL0) * EMB_DIM,
                                           128 * EMB_DIM)], sout1)
        pltpu.sync_copy(cats_hbm.at[f], rowb)

        def add_off(k, _):
            rowb[pl.ds(k * 16, 16)] = rowb[pl.ds(k * 16, 16)] + f * VPAD
            return 0

        lax.fori_loop(0, BATCH // 16, add_off, 0, unroll=8)
        pltpu.sync_copy(rowb, lcats_hbm.at[pl.ds(f * BATCH, BATCH)])
        pltpu.make_async_copy(outb0.at[pl.ds(0, w * EMB_DIM)],
                              ltab_hbm.at[pl.ds(0, w * EMB_DIM)],
                              sout0).wait()
        pltpu.make_async_copy(outb1.at[pl.ds(0, 128 * EMB_DIM)],
                              ltab_hbm.at[pl.ds(0, 128 * EMB_DIM)],
                              sout1).wait()


_reformat = functools.partial(
    pl.kernel,
    out_type=(
        jax.ShapeDtypeStruct((N_FIELDS * VPAD * EMB_DIM,), jnp.float32),
        jax.ShapeDtypeStruct((N_FIELDS * BATCH,), jnp.int32),
    ),
    mesh=plsc.VectorSubcoreMesh(core_axis_name="c", subcore_axis_name="s"),
    scratch_types=[
        pltpu.VMEM((EMB_DIM, 1024), jnp.float32),
        pltpu.VMEM((EMB_DIM, 1024), jnp.float32),
        pltpu.VMEM((1024 * EMB_DIM,), jnp.float32),
        pltpu.VMEM((1024 * EMB_DIM,), jnp.float32),
        pltpu.VMEM((BATCH,), jnp.int32),
        pltpu.VMEM((16, 128), jnp.float32),
        pltpu.SemaphoreType.DMA,
        pltpu.SemaphoreType.DMA,
        pltpu.SemaphoreType.DMA,
        pltpu.SemaphoreType.DMA,
    ],
    compiler_params=pltpu.CompilerParams(needs_layout_passes=False),
)(_reformat_body)


def _gather_body(lcats_hbm, ltab_hbm, out_hbm, idx_v, rows_v, sem):
    wid = lax.axis_index("s") * 2 + lax.axis_index("c")

    def body(t, carry):
        task = wid * TASKS_PER_TILE + t
        f = task // N_CHUNKS
        c = task % N_CHUNKS
        base = c * CHUNK
        pltpu.sync_copy(lcats_hbm.at[pl.ds(f * BATCH + base, CHUNK)], idx_v)
        pltpu.async_copy(ltab_hbm.at[idx_v], rows_v, sem).wait()
        pltpu.sync_copy(rows_v,
                        out_hbm.at[f // 8, pl.ds(base, CHUNK),
                                   pl.ds((f % 8) * EMB_DIM, EMB_DIM)])
        return carry

    lax.fori_loop(0, TASKS_PER_TILE, body, 0)


_gather = functools.partial(
    pl.kernel,
    out_type=jax.ShapeDtypeStruct((4, BATCH, 128), jnp.float32),
    mesh=plsc.VectorSubcoreMesh(core_axis_name="c", subcore_axis_name="s"),
    scratch_types=[
        pltpu.VMEM((CHUNK,), jnp.int32),
        pltpu.VMEM((CHUNK, EMB_DIM), jnp.float32),
        pltpu.SemaphoreType.DMA,
    ],
    compiler_params=pltpu.CompilerParams(use_tc_tiling_on_sc=False),
)(_gather_body)


def _mlp_body(cat_ref, nums_ref, w1e_ref, w1n_ref, b1_ref, w2_ref, b2_ref,
              out_ref):
    x4 = cat_ref[...]
    w1e = w1e_ref[...]
    acc = nums_ref[...] @ w1n_ref[...] + b1_ref[...]
    for g in range(3):
        acc = acc + x4[g] @ w1e[g * 128:(g + 1) * 128]
    acc = acc + x4[3][:, :32] @ w1e[384:416]
    h = jnp.maximum(acc, 0.0)
    out_ref[...] = h @ w2_ref[...] + b2_ref[...]


def _mlp(cats4, nums, w1e, w1n, b1, w2, b2):
    bm = 2048
    grid = BATCH // bm
    return pl.pallas_call(
        _mlp_body,
        grid=(grid,),
        in_specs=[
            pl.BlockSpec((4, bm, 128), lambda i: (0, i, 0)),
            pl.BlockSpec((bm, NUM_DIM), lambda i: (i, 0)),
            pl.BlockSpec((N_FIELDS * EMB_DIM, 64), lambda i: (0, 0)),
            pl.BlockSpec((NUM_DIM, 64), lambda i: (0, 0)),
            pl.BlockSpec((1, 64), lambda i: (0, 0)),
            pl.BlockSpec((64, 1), lambda i: (0, 0)),
            pl.BlockSpec((1, 1), lambda i: (0, 0)),
        ],
        out_specs=pl.BlockSpec((bm, 1), lambda i: (i, 0)),
        out_shape=jax.ShapeDtypeStruct((BATCH, 1), jnp.float32),
    )(cats4, nums, w1e, w1n, b1, w2, b2)


def kernel(cats, nums, emb_tables, W1, b1, W2, b2):
    tab_t = jnp.transpose(emb_tables, (0, 2, 1))
    tail = _tail(tab_t)
    ltab1, lcats = _reformat(tab_t, tail, cats.astype(jnp.int32))
    ltab = ltab1.reshape(N_FIELDS * VPAD, EMB_DIM)
    cats4 = _gather(lcats, ltab)
    w1e = W1[: N_FIELDS * EMB_DIM]
    w1n = W1[N_FIELDS * EMB_DIM :]
    return _mlp(cats4, nums, w1e, w1n, b1[None, :], W2, b2[None, :])


# speedup vs baseline: 12.6901x; 1.5600x over previous
import functools

import jax
import jax.numpy as jnp
from jax import lax
from jax.experimental import pallas as pl
from jax.experimental.pallas import tpu as pltpu
from jax.experimental.pallas import tpu_sc as plsc

N_FIELDS = 26
VOCAB = 100000
EMB_DIM = 16
NUM_DIM = 13
BATCH = 16384

VPAD = 100096
TAIL0 = 99968

NUM_WORKERS = 32
N_CHUNKS = 16
CHUNK = BATCH // N_CHUNKS
TASKS_PER_TILE = (N_FIELDS * N_CHUNKS) // NUM_WORKERS

BULK_CHUNKS = 97
TOT_BULK = N_FIELDS * BULK_CHUNKS
BULK_STEPS = (TOT_BULK + NUM_WORKERS - 1) // NUM_WORKERS


def _tail_body(tab_ref, out_ref):
    x = tab_ref[0]
    eye = jnp.eye(EMB_DIM, dtype=jnp.float32)
    y = jax.lax.dot_general(x, eye, (((0,), (0,)), ((), ())),
                            preferred_element_type=jnp.float32)
    y8 = y.reshape(16, 8, EMB_DIM)
    out_ref[...] = jnp.concatenate([y8[:, h, :] for h in range(8)], axis=1)


def _tail(tab_t):
    return pl.pallas_call(
        _tail_body,
        grid=(N_FIELDS,),
        in_specs=[pl.BlockSpec((1, EMB_DIM, 128),
                               lambda f: (f, 0, TAIL0 // 128))],
        out_specs=pl.BlockSpec((16, 128), lambda f: (f, 0)),
        out_shape=jax.ShapeDtypeStruct((N_FIELDS * 16, 128), jnp.float32),
    )(tab_t)


def _reformat_body(tab_hbm, tail_hbm, cats_hbm, ltab_hbm, lcats_hbm,
                   inb0, inb1, outb0, outb1, rowb, tailb,
                   sin0, sin1, sout0, sout1):
    wid = lax.axis_index("s") * 2 + lax.axis_index("c")
    lanes = lax.iota(jnp.int32, 16)

    def chunk_q(t):
        return jnp.minimum(t * NUM_WORKERS + wid, TOT_BULK - 1)

    def start_in(t, ib, sem):
        q = chunk_q(t)
        f = q // BULK_CHUNKS
        col0 = (q % BULK_CHUNKS) * 1024
        pltpu.async_copy(tab_hbm.at[f, :, pl.ds(col0, 1024)], ib, sem)

    def wait_in(sem):
        pltpu.make_async_copy(tab_hbm.at[0, :, pl.ds(0, 1024)], inb0,
                              sem).wait()

    def extract(ib, ob, width):
        @plsc.parallel_loop(0, width, unroll=8)
        def _(v):
            x = plsc.load_gather(ib, [lanes, jnp.full((16,), v, jnp.int32)])
            ob[pl.ds(v * EMB_DIM, EMB_DIM)] = x

    def start_out(t, ob, sem):
        q = chunk_q(t)
        f = q // BULK_CHUNKS
        col0 = (q % BULK_CHUNKS) * 1024
        dst = (f * VPAD + col0) * EMB_DIM
        pltpu.async_copy(ob, ltab_hbm.at[pl.ds(dst, 1024 * EMB_DIM)], sem)

    def wait_out(sem):
        pltpu.make_async_copy(outb0, ltab_hbm.at[pl.ds(0, 1024 * EMB_DIM)],
                              sem).wait()

    start_in(0, inb0, sin0)

    def phase(t, ib, ob, sin, sout, ib_next, sin_next):
        @pl.when(t + 1 < BULK_STEPS)
        def _():
            start_in(t + 1, ib_next, sin_next)

        wait_in(sin)

        @pl.when(t >= 2)
        def _():
            wait_out(sout)

        extract(ib, ob, 1024)
        start_out(t, ob, sout)

    def body(t, _):
        @pl.when(t % 2 == 0)
        def _():
            phase(t, inb0, outb0, sin0, sout0, inb1, sin1)

        @pl.when(t % 2 == 1)
        def _():
            phase(t, inb1, outb1, sin1, sout1, inb0, sin0)

        return 0

    lax.fori_loop(0, BULK_STEPS, body, 0)
    wait_out(sout0)
    wait_out(sout1)

    @pl.when(wid < N_FIELDS)
    def _():
        f = wid
        w = TAIL0 - BULK_CHUNKS * 1024
        pltpu.async_copy(tab_hbm.at[f, :, pl.ds(BULK_CHUNKS * 1024, w)],
                         inb0.at[:, pl.ds(0, w)], sin0)
        pltpu.make_async_copy(tab_hbm.at[0, :, pl.ds(0, w)],
                              inb0.at[:, pl.ds(0, w)], sin0).wait()
        extract(inb0, outb0, w)
        pltpu.async_copy(outb0.at[pl.ds(0, w * EMB_DIM)],
                         ltab_hbm.at[pl.ds((f * VPAD + BULK_CHUNKS * 1024)
                                           * EMB_DIM, w * EMB_DIM)], sout0)
        pltpu.sync_copy(tail_hbm.at[pl.ds(f * 16, 16)], tailb)

        def trow(v, _):
            x = plsc.load_gather(
                tailb, [jnp.full((16,), v // 8, jnp.int32),
                        (v % 8) * EMB_DIM + lanes])
            outb1[pl.ds(v * EMB_DIM, EMB_DIM)] = x
            return 0

        lax.fori_loop(0, 128, trow, 0, unroll=8)
        pltpu.async_copy(outb1.at[pl.ds(0, 128 * EMB_DIM)],
                         ltab_hbm.at[pl.ds((f * VPAD + TAIL0) * EMB_DIM,
                                           128 * EMB_DIM)], sout1)
        pltpu.sync_copy(cats_hbm.at[f], rowb)

        def add_off(k, _):
            rowb[pl.ds(k * 16, 16)] = rowb[pl.ds(k * 16, 16)] + f * VPAD
            return 0

        lax.fori_loop(0, BATCH // 16, add_off, 0, unroll=8)
        pltpu.sync_copy(rowb, lcats_hbm.at[pl.ds(f * BATCH, BATCH)])
        pltpu.make_async_copy(outb0.at[pl.ds(0, w * EMB_DIM)],
                              ltab_hbm.at[pl.ds(0, w * EMB_DIM)],
                              sout0).wait()
        pltpu.make_async_copy(outb1.at[pl.ds(0, 128 * EMB_DIM)],
                              ltab_hbm.at[pl.ds(0, 128 * EMB_DIM)],
                              sout1).wait()


_reformat = functools.partial(
    pl.kernel,
    out_type=(
        jax.ShapeDtypeStruct((N_FIELDS * VPAD * EMB_DIM,), jnp.float32),
        jax.ShapeDtypeStruct((N_FIELDS * BATCH,), jnp.int32),
    ),
    mesh=plsc.VectorSubcoreMesh(core_axis_name="c", subcore_axis_name="s"),
    scratch_types=[
        pltpu.VMEM((EMB_DIM, 1024), jnp.float32),
        pltpu.VMEM((EMB_DIM, 1024), jnp.float32),
        pltpu.VMEM((1024 * EMB_DIM,), jnp.float32),
        pltpu.VMEM((1024 * EMB_DIM,), jnp.float32),
        pltpu.VMEM((BATCH,), jnp.int32),
        pltpu.VMEM((16, 128), jnp.float32),
        pltpu.SemaphoreType.DMA,
        pltpu.SemaphoreType.DMA,
        pltpu.SemaphoreType.DMA,
        pltpu.SemaphoreType.DMA,
    ],
    compiler_params=pltpu.CompilerParams(needs_layout_passes=False),
)(_reformat_body)


def _gather_body(lcats_hbm, ltab_hbm, out_hbm, idx_v, rows_v, sem):
    wid = lax.axis_index("s") * 2 + lax.axis_index("c")

    def body(t, carry):
        task = wid * TASKS_PER_TILE + t
        f = task // N_CHUNKS
        c = task % N_CHUNKS
        base = c * CHUNK
        pltpu.sync_copy(lcats_hbm.at[pl.ds(f * BATCH + base, CHUNK)], idx_v)
        pltpu.async_copy(ltab_hbm.at[idx_v], rows_v, sem).wait()
        pltpu.sync_copy(rows_v,
                        out_hbm.at[f // 8, pl.ds(base, CHUNK),
                                   pl.ds((f % 8) * EMB_DIM, EMB_DIM)])
        return carry

    lax.fori_loop(0, TASKS_PER_TILE, body, 0)


_gather = functools.partial(
    pl.kernel,
    out_type=jax.ShapeDtypeStruct((4, BATCH, 128), jnp.float32),
    mesh=plsc.VectorSubcoreMesh(core_axis_name="c", subcore_axis_name="s"),
    scratch_types=[
        pltpu.VMEM((CHUNK,), jnp.int32),
        pltpu.VMEM((CHUNK, EMB_DIM), jnp.float32),
        pltpu.SemaphoreType.DMA,
    ],
    compiler_params=pltpu.CompilerParams(use_tc_tiling_on_sc=False),
)(_gather_body)


def _mlp_body(cat_ref, nums_ref, w1e_ref, w1n_ref, b1_ref, w2_ref, b2_ref,
              out_ref):
    x4 = cat_ref[...]
    w1e = w1e_ref[...]
    acc = nums_ref[...] @ w1n_ref[...] + b1_ref[...]
    for g in range(3):
        acc = acc + x4[g] @ w1e[g * 128:(g + 1) * 128]
    acc = acc + x4[3][:, :32] @ w1e[384:416]
    h = jnp.maximum(acc, 0.0)
    out_ref[...] = h @ w2_ref[...] + b2_ref[...]


def _mlp(cats4, nums, w1e, w1n, b1, w2, b2):
    bm = 2048
    grid = BATCH // bm
    return pl.pallas_call(
        _mlp_body,
        grid=(grid,),
        in_specs=[
            pl.BlockSpec((4, bm, 128), lambda i: (0, i, 0)),
            pl.BlockSpec((bm, NUM_DIM), lambda i: (i, 0)),
            pl.BlockSpec((N_FIELDS * EMB_DIM, 64), lambda i: (0, 0)),
            pl.BlockSpec((NUM_DIM, 64), lambda i: (0, 0)),
            pl.BlockSpec((1, 64), lambda i: (0, 0)),
            pl.BlockSpec((64, 1), lambda i: (0, 0)),
            pl.BlockSpec((1, 1), lambda i: (0, 0)),
        ],
        out_specs=pl.BlockSpec((bm, 1), lambda i: (i, 0)),
        out_shape=jax.ShapeDtypeStruct((BATCH, 1), jnp.float32),
    )(cats4, nums, w1e, w1n, b1, w2, b2)


def kernel(cats, nums, emb_tables, W1, b1, W2, b2):
    tab_t = jnp.transpose(emb_tables, (0, 2, 1))
    tail = _tail(tab_t)
    ltab1, lcats = _reformat(tab_t, tail, cats.astype(jnp.int32))
    ltab = ltab1.reshape(N_FIELDS * VPAD, EMB_DIM)
    cats4 = _gather(lcats, ltab)
    w1e = W1[: N_FIELDS * EMB_DIM]
    w1n = W1[N_FIELDS * EMB_DIM :]
    return _mlp(cats4, nums, w1e, w1n, b1[None, :], W2, b2[None, :])


# speedup vs baseline: 13.1826x; 1.0388x over previous
import functools

import jax
import jax.numpy as jnp
from jax import lax
from jax.experimental import pallas as pl
from jax.experimental.pallas import tpu as pltpu
from jax.experimental.pallas import tpu_sc as plsc

N_FIELDS = 26
VOCAB = 100000
EMB_DIM = 16
NUM_DIM = 13
BATCH = 16384

VPAD = 100096
TAIL0 = 99968

NUM_WORKERS = 32
N_CHUNKS = 16
CHUNK = BATCH // N_CHUNKS
TASKS_PER_TILE = (N_FIELDS * N_CHUNKS) // NUM_WORKERS

BULK_CHUNKS = 97
TOT_BULK = N_FIELDS * BULK_CHUNKS
BULK_STEPS = (TOT_BULK + NUM_WORKERS - 1) // NUM_WORKERS


def _tail_body(tab_ref, out_ref):
    x = tab_ref[0]
    eye = jnp.eye(EMB_DIM, dtype=jnp.float32)
    y = jax.lax.dot_general(x, eye, (((0,), (0,)), ((), ())),
                            preferred_element_type=jnp.float32)
    y8 = y.reshape(16, 8, EMB_DIM)
    out_ref[...] = jnp.concatenate([y8[:, h, :] for h in range(8)], axis=1)


def _tail(tab_t):
    return pl.pallas_call(
        _tail_body,
        grid=(N_FIELDS,),
        in_specs=[pl.BlockSpec((1, EMB_DIM, 128),
                               lambda f: (f, 0, TAIL0 // 128))],
        out_specs=pl.BlockSpec((16, 128), lambda f: (f, 0)),
        out_shape=jax.ShapeDtypeStruct((N_FIELDS * 16, 128), jnp.float32),
    )(tab_t)


def _reformat_body(tab_hbm, tail_hbm, cats_hbm, ltab_hbm, lcats_hbm,
                   inb0, inb1, outb0, outb1, rowb, tailb,
                   sin0, sin1, sout0, sout1):
    wid = lax.axis_index("s") * 2 + lax.axis_index("c")
    lanes = lax.iota(jnp.int32, 16)

    def chunk_q(t):
        return jnp.minimum(t * NUM_WORKERS + wid, TOT_BULK - 1)

    def start_in(t, ib, sem):
        q = chunk_q(t)
        f = q // BULK_CHUNKS
        col0 = (q % BULK_CHUNKS) * 1024
        pltpu.async_copy(tab_hbm.at[f, :, pl.ds(col0, 1024)], ib, sem)

    def wait_in(sem):
        pltpu.make_async_copy(tab_hbm.at[0, :, pl.ds(0, 1024)], inb0,
                              sem).wait()

    def extract(ib, ob, width):
        @plsc.parallel_loop(0, width, unroll=16)
        def _(v):
            x = plsc.load_gather(ib, [lanes, jnp.full((16,), v, jnp.int32)])
            ob[pl.ds(v * EMB_DIM, EMB_DIM)] = x

    def start_out(t, ob, sem):
        q = chunk_q(t)
        f = q // BULK_CHUNKS
        col0 = (q % BULK_CHUNKS) * 1024
        dst = (f * VPAD + col0) * EMB_DIM
        pltpu.async_copy(ob, ltab_hbm.at[pl.ds(dst, 1024 * EMB_DIM)], sem)

    def wait_out(sem):
        pltpu.make_async_copy(outb0, ltab_hbm.at[pl.ds(0, 1024 * EMB_DIM)],
                              sem).wait()

    start_in(0, inb0, sin0)

    def phase(t, ib, ob, sin, sout, ib_next, sin_next):
        @pl.when(t + 1 < BULK_STEPS)
        def _():
            start_in(t + 1, ib_next, sin_next)

        wait_in(sin)

        @pl.when(t >= 2)
        def _():
            wait_out(sout)

        extract(ib, ob, 1024)
        start_out(t, ob, sout)

    def body(t, _):
        @pl.when(t % 2 == 0)
        def _():
            phase(t, inb0, outb0, sin0, sout0, inb1, sin1)

        @pl.when(t % 2 == 1)
        def _():
            phase(t, inb1, outb1, sin1, sout1, inb0, sin0)

        return 0

    lax.fori_loop(0, BULK_STEPS, body, 0)
    wait_out(sout0)
    wait_out(sout1)

    @pl.when(wid < N_FIELDS)
    def _():
        f = wid
        w = TAIL0 - BULK_CHUNKS * 1024
        pltpu.async_copy(tab_hbm.at[f, :, pl.ds(BULK_CHUNKS * 1024, w)],
                         inb0.at[:, pl.ds(0, w)], sin0)
        pltpu.make_async_copy(tab_hbm.at[0, :, pl.ds(0, w)],
                              inb0.at[:, pl.ds(0, w)], sin0).wait()
        extract(inb0, outb0, w)
        pltpu.async_copy(outb0.at[pl.ds(0, w * EMB_DIM)],
                         ltab_hbm.at[pl.ds((f * VPAD + BULK_CHUNKS * 1024)
                                           * EMB_DIM, w * EMB_DIM)], sout0)
        pltpu.sync_copy(tail_hbm.at[pl.ds(f * 16, 16)], tailb)

        def trow(v, _):
            x = plsc.load_gather(
                tailb, [jnp.full((16,), v // 8, jnp.int32),
                        (v % 8) * EMB_DIM + lanes])
            outb1[pl.ds(v * EMB_DIM, EMB_DIM)] = x
            return 0

        lax.fori_loop(0, 128, trow, 0, unroll=8)
        pltpu.async_copy(outb1.at[pl.ds(0, 128 * EMB_DIM)],
                         ltab_hbm.at[pl.ds((f * VPAD + TAIL0) * EMB_DIM,
                                           128 * EMB_DIM)], sout1)
        pltpu.sync_copy(cats_hbm.at[f], rowb)

        def add_off(k, _):
            rowb[pl.ds(k * 16, 16)] = rowb[pl.ds(k * 16, 16)] + f * VPAD
            return 0

        lax.fori_loop(0, BATCH // 16, add_off, 0, unroll=8)
        pltpu.sync_copy(rowb, lcats_hbm.at[pl.ds(f * BATCH, BATCH)])
        pltpu.make_async_copy(outb0.at[pl.ds(0, w * EMB_DIM)],
                              ltab_hbm.at[pl.ds(0, w * EMB_DIM)],
                              sout0).wait()
        pltpu.make_async_copy(outb1.at[pl.ds(0, 128 * EMB_DIM)],
                              ltab_hbm.at[pl.ds(0, 128 * EMB_DIM)],
                              sout1).wait()


_reformat = functools.partial(
    pl.kernel,
    out_type=(
        jax.ShapeDtypeStruct((N_FIELDS * VPAD * EMB_DIM,), jnp.float32),
        jax.ShapeDtypeStruct((N_FIELDS * BATCH,), jnp.int32),
    ),
    mesh=plsc.VectorSubcoreMesh(core_axis_name="c", subcore_axis_name="s"),
    scratch_types=[
        pltpu.VMEM((EMB_DIM, 1024), jnp.float32),
        pltpu.VMEM((EMB_DIM, 1024), jnp.float32),
        pltpu.VMEM((1024 * EMB_DIM,), jnp.float32),
        pltpu.VMEM((1024 * EMB_DIM,), jnp.float32),
        pltpu.VMEM((BATCH,), jnp.int32),
        pltpu.VMEM((16, 128), jnp.float32),
        pltpu.SemaphoreType.DMA,
        pltpu.SemaphoreType.DMA,
        pltpu.SemaphoreType.DMA,
        pltpu.SemaphoreType.DMA,
    ],
    compiler_params=pltpu.CompilerParams(needs_layout_passes=False),
)(_reformat_body)


def _gather_body(lcats_hbm, ltab_hbm, out_hbm, idx_v, rows_v, sem):
    wid = lax.axis_index("s") * 2 + lax.axis_index("c")

    def body(t, carry):
        task = wid * TASKS_PER_TILE + t
        f = task // N_CHUNKS
        c = task % N_CHUNKS
        base = c * CHUNK
        pltpu.sync_copy(lcats_hbm.at[pl.ds(f * BATCH + base, CHUNK)], idx_v)
        pltpu.async_copy(ltab_hbm.at[idx_v], rows_v, sem).wait()
        pltpu.sync_copy(rows_v,
                        out_hbm.at[f // 8, pl.ds(base, CHUNK),
                                   pl.ds((f % 8) * EMB_DIM, EMB_DIM)])
        return carry

    lax.fori_loop(0, TASKS_PER_TILE, body, 0)


_gather = functools.partial(
    pl.kernel,
    out_type=jax.ShapeDtypeStruct((4, BATCH, 128), jnp.float32),
    mesh=plsc.VectorSubcoreMesh(core_axis_name="c", subcore_axis_name="s"),
    scratch_types=[
        pltpu.VMEM((CHUNK,), jnp.int32),
        pltpu.VMEM((CHUNK, EMB_DIM), jnp.float32),
        pltpu.SemaphoreType.DMA,
    ],
    compiler_params=pltpu.CompilerParams(use_tc_tiling_on_sc=False),
)(_gather_body)


def _mlp_body(cat_ref, nums_ref, w1e_ref, w1n_ref, b1_ref, w2_ref, b2_ref,
              out_ref):
    x4 = cat_ref[...]
    w1e = w1e_ref[...]
    acc = nums_ref[...] @ w1n_ref[...] + b1_ref[...]
    for g in range(3):
        acc = acc + x4[g] @ w1e[g * 128:(g + 1) * 128]
    acc = acc + x4[3][:, :32] @ w1e[384:416]
    h = jnp.maximum(acc, 0.0)
    out_ref[...] = h @ w2_ref[...] + b2_ref[...]


def _mlp(cats4, nums, w1e, w1n, b1, w2, b2):
    bm = 2048
    grid = BATCH // bm
    return pl.pallas_call(
        _mlp_body,
        grid=(grid,),
        in_specs=[
            pl.BlockSpec((4, bm, 128), lambda i: (0, i, 0)),
            pl.BlockSpec((bm, NUM_DIM), lambda i: (i, 0)),
            pl.BlockSpec((N_FIELDS * EMB_DIM, 64), lambda i: (0, 0)),
            pl.BlockSpec((NUM_DIM, 64), lambda i: (0, 0)),
            pl.BlockSpec((1, 64), lambda i: (0, 0)),
            pl.BlockSpec((64, 1), lambda i: (0, 0)),
            pl.BlockSpec((1, 1), lambda i: (0, 0)),
        ],
        out_specs=pl.BlockSpec((bm, 1), lambda i: (i, 0)),
        out_shape=jax.ShapeDtypeStruct((BATCH, 1), jnp.float32),
    )(cats4, nums, w1e, w1n, b1, w2, b2)


def kernel(cats, nums, emb_tables, W1, b1, W2, b2):
    tab_t = jnp.transpose(emb_tables, (0, 2, 1))
    tail = _tail(tab_t)
    ltab1, lcats = _reformat(tab_t, tail, cats.astype(jnp.int32))
    ltab = ltab1.reshape(N_FIELDS * VPAD, EMB_DIM)
    cats4 = _gather(lcats, ltab)
    w1e = W1[: N_FIELDS * EMB_DIM]
    w1n = W1[N_FIELDS * EMB_DIM :]
    return _mlp(cats4, nums, w1e, w1n, b1[None, :], W2, b2[None, :])
